# Initial kernel scaffold; baseline (speedup 1.0000x reference)
import functools

import jax
import jax.numpy as jnp
from jax import lax
from jax.experimental import pallas as pl
from jax.experimental.pallas import tpu as pltpu
from jax.experimental.pallas import tpu_sc as plsc

NUM_EMBED = 1000000
EMBED_DIM = 64
BATCH = 16384
HIST = 50
B_TOTAL = BATCH * HIST

_info = plsc.get_sparse_core_info()
NC, NS = _info.num_cores, _info.num_subcores
NW = NC * NS
B_PER_W = B_TOTAL // NW
CHUNK = 128
NCHUNK = B_PER_W // CHUNK


def _make_kernel():
    mesh = plsc.VectorSubcoreMesh(core_axis_name="c", subcore_axis_name="s")

    @functools.partial(
        pl.kernel,
        mesh=mesh,
        out_type=jax.ShapeDtypeStruct((B_TOTAL, EMBED_DIM), jnp.float32),
        scratch_types=[
            pltpu.VMEM((NCHUNK, CHUNK), jnp.int32),
            pltpu.VMEM((CHUNK, EMBED_DIM), jnp.float32),
            pltpu.SemaphoreType.DMA,
        ],
    )
    def k(table_hbm, idx_hbm, out_hbm, idx_v, rows_v, sem):
        wid = lax.axis_index("s") * NC + lax.axis_index("c")
        base = wid * B_PER_W
        pltpu.sync_copy(idx_hbm.at[wid], idx_v)

        def body(j, carry):
            pltpu.async_copy(table_hbm.at[idx_v.at[j]], rows_v, sem).wait()
            pltpu.sync_copy(rows_v, out_hbm.at[pl.ds(base + j * CHUNK, CHUNK)])
            return carry

        lax.fori_loop(0, NCHUNK, body, 0)

    return k


_sc_gather = _make_kernel()


def kernel(inputs, vec_matrix):
    idx = inputs.reshape(NW, NCHUNK, CHUNK).astype(jnp.int32)
    out = _sc_gather(vec_matrix, idx)
    return out.reshape(BATCH, HIST, EMBED_DIM)


# speedup vs baseline: 1.6849x; 1.6849x over previous
import functools

import jax
import jax.numpy as jnp
from jax import lax
from jax.experimental import pallas as pl
from jax.experimental.pallas import tpu as pltpu
from jax.experimental.pallas import tpu_sc as plsc

NUM_EMBED = 1000000
EMBED_DIM = 64
BATCH = 16384
HIST = 50
B_TOTAL = BATCH * HIST

_info = plsc.get_sparse_core_info()
NC, NS = _info.num_cores, _info.num_subcores
NW = NC * NS
B_PER_W = B_TOTAL // NW
CHUNK = 128
NCHUNK = B_PER_W // CHUNK


def _make_kernel():
    mesh = plsc.VectorSubcoreMesh(core_axis_name="c", subcore_axis_name="s")

    @functools.partial(
        pl.kernel,
        mesh=mesh,
        out_type=jax.ShapeDtypeStruct((B_TOTAL, EMBED_DIM), jnp.float32),
        compiler_params=pltpu.CompilerParams(use_tc_tiling_on_sc=False),
        scratch_types=[
            pltpu.VMEM((NCHUNK, CHUNK), jnp.int32),
            pltpu.VMEM((CHUNK, EMBED_DIM), jnp.float32),
            pltpu.SemaphoreType.DMA,
        ],
    )
    def k(table_hbm, idx_hbm, out_hbm, idx_v, rows_v, sem):
        wid = lax.axis_index("s") * NC + lax.axis_index("c")
        base = wid * B_PER_W
        pltpu.sync_copy(idx_hbm.at[wid], idx_v)

        def body(j, carry):
            pltpu.async_copy(table_hbm.at[idx_v.at[j]], rows_v, sem).wait()
            pltpu.sync_copy(rows_v, out_hbm.at[pl.ds(base + j * CHUNK, CHUNK)])
            return carry

        lax.fori_loop(0, NCHUNK, body, 0)

    return k


_sc_gather = _make_kernel()


def kernel(inputs, vec_matrix):
    idx = inputs.reshape(NW, NCHUNK, CHUNK).astype(jnp.int32)
    out = _sc_gather(vec_matrix, idx)
    return out.reshape(BATCH, HIST, EMBED_DIM)


# speedup vs baseline: 1.8762x; 1.1136x over previous
import functools

import jax
import jax.numpy as jnp
from jax import lax
from jax.experimental import pallas as pl
from jax.experimental.pallas import tpu as pltpu
from jax.experimental.pallas import tpu_sc as plsc

NUM_EMBED = 1000000
EMBED_DIM = 64
BATCH = 16384
HIST = 50
B_TOTAL = BATCH * HIST

_info = plsc.get_sparse_core_info()
NC, NS = _info.num_cores, _info.num_subcores
NW = NC * NS
B_PER_W = B_TOTAL // NW
CHUNK = 128
NCHUNK = B_PER_W // CHUNK
K = 4
GROUP_ROWS = K * CHUNK
NGROUP = NCHUNK // K


def _make_kernel():
    mesh = plsc.VectorSubcoreMesh(core_axis_name="c", subcore_axis_name="s")

    @functools.partial(
        pl.kernel,
        mesh=mesh,
        out_type=jax.ShapeDtypeStruct((B_TOTAL, EMBED_DIM), jnp.float32),
        compiler_params=pltpu.CompilerParams(use_tc_tiling_on_sc=False),
        scratch_types=[
            pltpu.VMEM((NCHUNK, CHUNK), jnp.int32),
            pltpu.VMEM((GROUP_ROWS, EMBED_DIM), jnp.float32),
            pltpu.VMEM((GROUP_ROWS, EMBED_DIM), jnp.float32),
            pltpu.SemaphoreType.DMA,
            pltpu.SemaphoreType.DMA,
            pltpu.SemaphoreType.DMA,
            pltpu.SemaphoreType.DMA,
        ],
    )
    def k(table_hbm, idx_hbm, out_hbm, idx_v, rows0, rows1, gs0, gs1, ps0, ps1):
        wid = lax.axis_index("s") * NC + lax.axis_index("c")
        base = wid * B_PER_W
        pltpu.sync_copy(idx_hbm.at[wid], idx_v)

        bufs = ((rows0, gs0, ps0), (rows1, gs1, ps1))

        def fire_gather(g, rows, gsem):
            for c in range(K):
                pltpu.async_copy(
                    table_hbm.at[idx_v.at[g * K + c]],
                    rows.at[pl.ds(c * CHUNK, CHUNK)],
                    gsem,
                )

        def drain_gather(rows, gsem):
            pltpu.make_async_copy(
                table_hbm.at[pl.ds(0, GROUP_ROWS)], rows, gsem
            ).wait()

        def fire_put(g, rows, psem):
            pltpu.async_copy(
                rows, out_hbm.at[pl.ds(base + g * GROUP_ROWS, GROUP_ROWS)], psem
            )

        def drain_put(rows, psem):
            pltpu.make_async_copy(
                table_hbm.at[pl.ds(0, GROUP_ROWS)], rows, psem
            ).wait()

        fire_gather(0, rows0, gs0)
        fire_gather(1, rows1, gs1)

        def body(t, carry):
            for phase in range(2):
                g = 2 * t + phase
                rows, gsem, psem = bufs[phase]
                drain_gather(rows, gsem)
                fire_put(g, rows, psem)
                drain_put(rows, psem)

                @pl.when(g + 2 < NGROUP)
                def _():
                    fire_gather(g + 2, rows, gsem)

            return carry

        lax.fori_loop(0, NGROUP // 2, body, 0)

    return k


_sc_gather = _make_kernel()


def kernel(inputs, vec_matrix):
    idx = inputs.reshape(NW, NCHUNK, CHUNK).astype(jnp.int32)
    out = _sc_gather(vec_matrix, idx)
    return out.reshape(BATCH, HIST, EMBED_DIM)
